# Initial kernel scaffold; baseline (speedup 1.0000x reference)
import jax
import jax.numpy as jnp
from jax.experimental import pallas as pl


def _body(x_ref, pe_ref, o_ref):
    o_ref[...] = x_ref[...] + pe_ref[:, None, :]


def kernel(x, pos_embed_weight):
    S, B, D = x.shape
    BS = 256
    return pl.pallas_call(
        _body,
        grid=(S // BS,),
        in_specs=[
            pl.BlockSpec((BS, B, D), lambda i: (i, 0, 0)),
            pl.BlockSpec((BS, D), lambda i: (i, 0)),
        ],
        out_specs=pl.BlockSpec((BS, B, D), lambda i: (i, 0, 0)),
        out_shape=jax.ShapeDtypeStruct((S, B, D), x.dtype),
    )(x, pos_embed_weight[:S])


# speedup vs baseline: 2.1881x; 2.1881x over previous
import jax
import jax.numpy as jnp
from jax.experimental import pallas as pl


def _body(x_ref, pe_ref, o_ref):
    o_ref[...] = x_ref[...] + pe_ref[...][:, None, :]


def kernel(x, pos_embed_weight):
    S, B, D = x.shape
    BS = 256
    return pl.pallas_call(
        _body,
        grid=(S // BS,),
        in_specs=[
            pl.BlockSpec((BS, B, D), lambda i: (i, 0, 0)),
            pl.BlockSpec((BS, D), lambda i: (i, 0)),
        ],
        out_specs=pl.BlockSpec((BS, B, D), lambda i: (i, 0, 0)),
        out_shape=jax.ShapeDtypeStruct((S, B, D), x.dtype),
    )(x, pos_embed_weight[:S])
